# baseline (speedup 1.0000x reference)
import functools

import jax
import jax.numpy as jnp
from jax import lax
from jax.experimental import pallas as pl
from jax.experimental.pallas import tpu as pltpu
from jax.experimental.pallas import tpu_sc as plsc

_NUM_CLASSES = 100000
_EMBED_DIM = 128
_BATCH = 16384
_NC = 2
_NS = 16
_L = 16
_NW = _NC * _NS
_B_PER_W = _BATCH // _NW
_NCHUNK = 16
_RC = _B_PER_W // _NCHUNK


def _rsqrt_newton(s):
    i = lax.bitcast_convert_type(s, jnp.int32)
    i = jnp.int32(0x5F3759DF) - lax.shift_right_logical(i, 1)
    y = lax.bitcast_convert_type(i, jnp.float32)
    y = y * (1.5 - 0.5 * s * y * y)
    return y


def _lane_shuffle(v, idx):
    dnums = lax.GatherDimensionNumbers(
        offset_dims=(), collapsed_slice_dims=(0,), start_index_map=(0,))
    return lax.gather(v, idx[:, None], dnums, slice_sizes=(1,),
                      mode=lax.GatherScatterMode.PROMISE_IN_BOUNDS)


def _allsum(v, lanes):
    for k in (8, 4, 2, 1):
        v = v + _lane_shuffle(v, lanes ^ k)
    return v


def _pack_sums2(a0, a1, lanes):
    h0 = a0 + _lane_shuffle(a0, lanes ^ 8)
    h1 = a1 + _lane_shuffle(a1, lanes ^ 8)
    m = jnp.where(lanes < 8, h0, h1)
    for k in (4, 2, 1):
        m = m + _lane_shuffle(m, lanes ^ k)
    return m


_mesh = plsc.VectorSubcoreMesh(core_axis_name="c", subcore_axis_name="s")


@functools.partial(
    pl.kernel,
    mesh=_mesh,
    out_type=jax.ShapeDtypeStruct((_BATCH, _EMBED_DIM), jnp.float32),
    scratch_types=[
        pltpu.VMEM((_B_PER_W,), jnp.int32),
        pltpu.VMEM((_B_PER_W, _EMBED_DIM), jnp.float32),
        pltpu.SemaphoreType.DMA,
        pltpu.SemaphoreType.DMA,
    ],
)
def _gather_normalize(idx_hbm, table_hbm, out_hbm, idx_v, rows_v, sem_g, sem_s):
    wid = lax.axis_index("s") * _NC + lax.axis_index("c")
    base = wid * _B_PER_W
    pltpu.sync_copy(idx_hbm.at[pl.ds(base, _B_PER_W)], idx_v)

    lanes = lax.iota(jnp.int32, _L)

    def _issue_gather(c, carry):
        pltpu.async_copy(
            table_hbm.at[idx_v.at[pl.ds(c * _RC, _RC)]],
            rows_v.at[pl.ds(c * _RC, _RC)], sem_g)
        return carry

    lax.fori_loop(0, _NCHUNK, _issue_gather, 0)

    def _chunk(c, carry):
        pltpu.make_async_copy(
            table_hbm.at[idx_v.at[pl.ds(0, _RC)]],
            rows_v.at[pl.ds(0, _RC)], sem_g).wait()
        row0 = c * _RC

        @plsc.parallel_loop(0, _RC)
        def _row(k):
            i = row0 + k
            vs = [rows_v[i, pl.ds(j * _L, _L)]
                  for j in range(_EMBED_DIM // _L)]
            sq = [v * v for v in vs]
            while len(sq) > 1:
                sq = [a + b for a, b in zip(sq[::2], sq[1::2])]
            s = _allsum(sq[0], lanes)
            scale = jnp.where(s >= 1e-24, _rsqrt_newton(s), 1e12)
            for j, v in enumerate(vs):
                rows_v[i, pl.ds(j * _L, _L)] = v * scale

        pltpu.async_copy(
            rows_v.at[pl.ds(row0, _RC)],
            out_hbm.at[pl.ds(base + row0, _RC)], sem_s)
        return carry

    lax.fori_loop(0, _NCHUNK, _chunk, 0)
    pltpu.make_async_copy(rows_v, out_hbm.at[pl.ds(base, _B_PER_W)],
                          sem_s).wait()


def kernel(indices, attributes):
    return _gather_normalize(indices.astype(jnp.int32), attributes)


# speedup vs baseline: 1.0107x; 1.0107x over previous
import functools

import jax
import jax.numpy as jnp
from jax import lax
from jax.experimental import pallas as pl
from jax.experimental.pallas import tpu as pltpu
from jax.experimental.pallas import tpu_sc as plsc

_NUM_CLASSES = 100000
_EMBED_DIM = 128
_BATCH = 16384
_NC = 2
_NS = 16
_L = 16
_NW = _NC * _NS
_B_PER_W = _BATCH // _NW
_NCHUNK = 8
_RC = _B_PER_W // _NCHUNK


def _rsqrt_newton(s):
    i = lax.bitcast_convert_type(s, jnp.int32)
    i = jnp.int32(0x5F3759DF) - lax.shift_right_logical(i, 1)
    y = lax.bitcast_convert_type(i, jnp.float32)
    y = y * (1.5 - 0.5 * s * y * y)
    return y


def _lane_shuffle(v, idx):
    dnums = lax.GatherDimensionNumbers(
        offset_dims=(), collapsed_slice_dims=(0,), start_index_map=(0,))
    return lax.gather(v, idx[:, None], dnums, slice_sizes=(1,),
                      mode=lax.GatherScatterMode.PROMISE_IN_BOUNDS)


def _allsum(v, lanes):
    for k in (8, 4, 2, 1):
        v = v + _lane_shuffle(v, lanes ^ k)
    return v


_mesh = plsc.VectorSubcoreMesh(core_axis_name="c", subcore_axis_name="s")


@functools.partial(
    pl.kernel,
    mesh=_mesh,
    out_type=jax.ShapeDtypeStruct((_BATCH, _EMBED_DIM), jnp.float32),
    scratch_types=[
        pltpu.VMEM((_B_PER_W,), jnp.int32),
        pltpu.VMEM((_B_PER_W, _EMBED_DIM), jnp.float32),
        pltpu.SemaphoreType.DMA,
        pltpu.SemaphoreType.DMA,
    ],
)
def _gather_normalize(idx_hbm, table_hbm, out_hbm, idx_v, rows_v, sem_g, sem_s):
    wid = lax.axis_index("s") * _NC + lax.axis_index("c")
    base = wid * _B_PER_W
    pltpu.sync_copy(idx_hbm.at[pl.ds(base, _B_PER_W)], idx_v)

    lanes = lax.iota(jnp.int32, _L)

    def _issue_gather(c, carry):
        pltpu.async_copy(
            table_hbm.at[idx_v.at[pl.ds(c * _RC, _RC)]],
            rows_v.at[pl.ds(c * _RC, _RC)], sem_g)
        return carry

    lax.fori_loop(0, _NCHUNK, _issue_gather, 0)

    def _chunk(c, carry):
        pltpu.make_async_copy(
            table_hbm.at[idx_v.at[pl.ds(0, _RC)]],
            rows_v.at[pl.ds(0, _RC)], sem_g).wait()
        row0 = c * _RC

        @plsc.parallel_loop(0, _RC)
        def _row(k):
            i = row0 + k
            vs = [rows_v[i, pl.ds(j * _L, _L)]
                  for j in range(_EMBED_DIM // _L)]
            sq = [v * v for v in vs]
            while len(sq) > 1:
                sq = [a + b for a, b in zip(sq[::2], sq[1::2])]
            s = _allsum(sq[0], lanes)
            scale = jnp.where(s >= 1e-24, _rsqrt_newton(s), 1e12)
            for j, v in enumerate(vs):
                rows_v[i, pl.ds(j * _L, _L)] = v * scale

        pltpu.async_copy(
            rows_v.at[pl.ds(row0, _RC)],
            out_hbm.at[pl.ds(base + row0, _RC)], sem_s)
        return carry

    lax.fori_loop(0, _NCHUNK, _chunk, 0)
    pltpu.make_async_copy(rows_v, out_hbm.at[pl.ds(base, _B_PER_W)],
                          sem_s).wait()


def kernel(indices, attributes):
    return _gather_normalize(indices.astype(jnp.int32), attributes)
